# Initial kernel scaffold; baseline (speedup 1.0000x reference)
import jax
import jax.numpy as jnp
from jax.experimental import pallas as pl


def kernel(sin_fan, fbp_filter, A_vals, scale, bias, A_rows, A_cols):
    raise NotImplementedError("write your pallas kernel here")


# speedup vs baseline: 35.2130x; 35.2130x over previous
import functools

import jax
import jax.numpy as jnp
from jax import lax
from jax.experimental import pallas as pl
from jax.experimental.pallas import tpu as pltpu
from jax.experimental.pallas import tpu_sc as plsc

B = 4
N_ANG = 180
N_DET = 384
SIN_SZ = N_ANG * N_DET
OUT_SZ = 256
NPIX = OUT_SZ * OUT_SZ
FILT_LEN = 129
HALF = (FILT_LEN - 1) // 2
PADDED = N_DET + 2 * HALF
NROW = B * N_ANG

NC = 2
NS = 16
NW = NC * NS
LANES = 16

PIX_PER_W = NPIX // NW
CHUNK_PIX = 128
NCHUNK = PIX_PER_W // CHUNK_PIX
CHUNK_NNZ = CHUNK_PIX * N_ANG


def _conv_body(xp_ref, k_ref, y_ref):
    y_ref[...] = jnp.dot(xp_ref[...], k_ref[...],
                         preferred_element_type=jnp.float32)


_conv = pl.pallas_call(
    _conv_body,
    out_shape=jax.ShapeDtypeStruct((NROW, N_DET), jnp.float32),
)


@functools.lru_cache(maxsize=1)
def _make_backproject():
    mesh = plsc.VectorSubcoreMesh(
        core_axis_name="c", subcore_axis_name="s",
        num_cores=NC, num_subcores=NS)

    @functools.partial(
        pl.kernel,
        mesh=mesh,
        compiler_params=pltpu.CompilerParams(needs_layout_passes=False),
        out_type=jax.ShapeDtypeStruct((B, NPIX), jnp.float32),
        scratch_types=[
            pltpu.VMEM((SIN_SZ,), jnp.float32),
            pltpu.VMEM((CHUNK_NNZ,), jnp.int32),
            pltpu.VMEM((CHUNK_NNZ,), jnp.float32),
            pltpu.VMEM((PIX_PER_W,), jnp.float32),
            pltpu.VMEM((LANES,), jnp.float32),
        ],
    )
    def backproject(sf_hbm, rows_hbm, vals_hbm, bias_hbm, out_hbm,
                    table_v, rows_v, vals_v, out_v, bias_v):
        wid = lax.axis_index("s") * NC + lax.axis_index("c")
        pltpu.sync_copy(bias_hbm, bias_v)
        bias_vec = bias_v[...]
        lane = lax.iota(jnp.int32, LANES)

        for b in range(B):
            pltpu.sync_copy(sf_hbm.at[b], table_v)

            def chunk_body(c, _):
                base = wid * (PIX_PER_W * N_ANG) + c * CHUNK_NNZ
                pltpu.sync_copy(rows_hbm.at[pl.ds(base, CHUNK_NNZ)], rows_v)
                pltpu.sync_copy(vals_hbm.at[pl.ds(base, CHUNK_NNZ)], vals_v)
                for pb in range(CHUNK_PIX // LANES):
                    idx0 = lane * N_ANG + pb * (LANES * N_ANG)

                    def k_body(k, carry):
                        acc, idxv = carry
                        rg = plsc.load_gather(rows_v, [idxv])
                        vg = plsc.load_gather(vals_v, [idxv])
                        tg = plsc.load_gather(table_v, [rg])
                        return acc + tg * vg, idxv + 1

                    acc, _ = lax.fori_loop(
                        0, N_ANG, k_body,
                        (jnp.zeros((LANES,), jnp.float32), idx0))
                    out_v[pl.ds(c * CHUNK_PIX + pb * LANES, LANES)] = (
                        acc + bias_vec)
                return 0

            lax.fori_loop(0, NCHUNK, chunk_body, 0)
            pltpu.sync_copy(
                out_v, out_hbm.at[b, pl.ds(wid * PIX_PER_W, PIX_PER_W)])

    return backproject


def kernel(sin_fan, fbp_filter, A_vals, scale, bias, A_rows, A_cols):
    x = sin_fan.reshape(NROW, N_DET)
    xp = jnp.pad(x, ((0, 0), (HALF, HALF)))
    f = fbp_filter.reshape(FILT_LEN) * scale[0]
    c_ix = jnp.arange(PADDED, dtype=jnp.int32)[:, None]
    d_ix = jnp.arange(N_DET, dtype=jnp.int32)[None, :]
    diff = c_ix - d_ix
    band = jnp.where((diff >= 0) & (diff < FILT_LEN),
                     f[jnp.clip(diff, 0, FILT_LEN - 1)], 0.0)
    sf = _conv(xp, band).reshape(B, SIN_SZ)

    bias16 = jnp.broadcast_to(bias.astype(jnp.float32), (LANES,))
    out = _make_backproject()(sf, A_rows, A_vals, bias16)
    return out.reshape(B, OUT_SZ, OUT_SZ, 1)


# speedup vs baseline: 44.3070x; 1.2583x over previous
import functools

import jax
import jax.numpy as jnp
from jax import lax
from jax.experimental import pallas as pl
from jax.experimental.pallas import tpu as pltpu
from jax.experimental.pallas import tpu_sc as plsc

B = 4
N_ANG = 180
N_DET = 384
SIN_SZ = N_ANG * N_DET
OUT_SZ = 256
NPIX = OUT_SZ * OUT_SZ
FILT_LEN = 129
HALF = (FILT_LEN - 1) // 2
PADDED = N_DET + 2 * HALF
NROW = B * N_ANG

NC = 2
NS = 16
NW = NC * NS
LANES = 16

PIX_PER_W = NPIX // NW
CHUNK_PIX = 64
NCHUNK = PIX_PER_W // CHUNK_PIX
CHUNK_NNZ = CHUNK_PIX * N_ANG
UNROLL = 6


def _conv_body(xp_ref, k_ref, y_ref):
    y_ref[...] = jnp.dot(xp_ref[...], k_ref[...],
                         preferred_element_type=jnp.float32)


_conv = pl.pallas_call(
    _conv_body,
    out_shape=jax.ShapeDtypeStruct((NROW, N_DET), jnp.float32),
)


@functools.lru_cache(maxsize=1)
def _make_backproject():
    mesh = plsc.VectorSubcoreMesh(
        core_axis_name="c", subcore_axis_name="s",
        num_cores=NC, num_subcores=NS)

    @functools.partial(
        pl.kernel,
        mesh=mesh,
        compiler_params=pltpu.CompilerParams(needs_layout_passes=False),
        out_type=jax.ShapeDtypeStruct((B, NPIX), jnp.float32),
        scratch_types=[
            pltpu.VMEM((SIN_SZ,), jnp.float32),
            pltpu.VMEM((CHUNK_NNZ,), jnp.int32),
            pltpu.VMEM((CHUNK_NNZ,), jnp.int32),
            pltpu.VMEM((CHUNK_NNZ,), jnp.float32),
            pltpu.VMEM((CHUNK_NNZ,), jnp.float32),
            pltpu.VMEM((PIX_PER_W,), jnp.float32),
            pltpu.VMEM((LANES,), jnp.float32),
            pltpu.SemaphoreType.DMA((2, 2)),
        ],
    )
    def backproject(sf_hbm, rows_hbm, vals_hbm, bias_hbm, out_hbm,
                    table_v, rows0_v, rows1_v, vals0_v, vals1_v,
                    out_v, bias_v, sems):
        wid = lax.axis_index("s") * NC + lax.axis_index("c")
        rows_bufs = (rows0_v, rows1_v)
        vals_bufs = (vals0_v, vals1_v)
        pltpu.sync_copy(bias_hbm, bias_v)
        bias_vec = bias_v[...]
        lane = lax.iota(jnp.int32, LANES)
        nnz0 = wid * (PIX_PER_W * N_ANG)

        def start(buf, c):
            base = nnz0 + (c & (NCHUNK - 1)) * CHUNK_NNZ
            pltpu.async_copy(rows_hbm.at[pl.ds(base, CHUNK_NNZ)],
                             rows_bufs[buf], sems.at[buf, 0])
            pltpu.async_copy(vals_hbm.at[pl.ds(base, CHUNK_NNZ)],
                             vals_bufs[buf], sems.at[buf, 1])

        def wait(buf):
            pltpu.make_async_copy(rows_hbm.at[pl.ds(0, CHUNK_NNZ)],
                                  rows_bufs[buf], sems.at[buf, 0]).wait()
            pltpu.make_async_copy(vals_hbm.at[pl.ds(0, CHUNK_NNZ)],
                                  vals_bufs[buf], sems.at[buf, 1]).wait()

        def compute(buf, c):
            rbuf = rows_bufs[buf]
            vbuf = vals_bufs[buf]
            for pb in range(CHUNK_PIX // LANES):
                idx0 = lane * N_ANG + pb * (LANES * N_ANG)

                def k_body(k, carry):
                    acc, idxv = carry
                    for u in range(UNROLL):
                        iu = idxv + u
                        rg = plsc.load_gather(rbuf, [iu])
                        vg = plsc.load_gather(vbuf, [iu])
                        tg = plsc.load_gather(table_v, [rg])
                        acc = acc + tg * vg
                    return acc, idxv + UNROLL

                acc, _ = lax.fori_loop(
                    0, N_ANG // UNROLL, k_body,
                    (jnp.zeros((LANES,), jnp.float32), idx0))
                out_v[pl.ds(c * CHUNK_PIX + pb * LANES, LANES)] = (
                    acc + bias_vec)

        start(0, 0)
        start(1, 1)
        for b in range(B):
            pltpu.sync_copy(sf_hbm.at[b], table_v)

            def pair_body(i, _):
                c0 = i * 2
                wait(0)
                compute(0, c0)
                start(0, c0 + 2)
                wait(1)
                compute(1, c0 + 1)
                start(1, c0 + 3)
                return 0

            lax.fori_loop(0, NCHUNK // 2, pair_body, 0)
            pltpu.sync_copy(
                out_v, out_hbm.at[b, pl.ds(wid * PIX_PER_W, PIX_PER_W)])
        wait(0)
        wait(1)

    return backproject


def kernel(sin_fan, fbp_filter, A_vals, scale, bias, A_rows, A_cols):
    x = sin_fan.reshape(NROW, N_DET)
    xp = jnp.pad(x, ((0, 0), (HALF, HALF)))
    f = fbp_filter.reshape(FILT_LEN) * scale[0]
    c_ix = jnp.arange(PADDED, dtype=jnp.int32)[:, None]
    d_ix = jnp.arange(N_DET, dtype=jnp.int32)[None, :]
    diff = c_ix - d_ix
    band = jnp.where((diff >= 0) & (diff < FILT_LEN),
                     f[jnp.clip(diff, 0, FILT_LEN - 1)], 0.0)
    sf = _conv(xp, band).reshape(B, SIN_SZ)

    bias16 = jnp.broadcast_to(bias.astype(jnp.float32), (LANES,))
    out = _make_backproject()(sf, A_rows, A_vals, bias16)
    return out.reshape(B, OUT_SZ, OUT_SZ, 1)
